# baseline (speedup 1.0000x reference)
import functools

import jax
import jax.numpy as jnp
from jax.experimental import pallas as pl
from jax.experimental.pallas import tpu as pltpu

_N = 25274
_DIM = 16
_K = 4
_B = 64

_KT = 2048
_T = 1024
_C = 256

_SELU_ALPHA = 1.6732632423543772
_SELU_SCALE = 1.0507009873554805


def _selu(v):
    return _SELU_SCALE * jnp.where(v > 0, v, _SELU_ALPHA * (jnp.exp(v) - 1.0))


_DNT = (((1,), (1,)), ((), ()))


def _enc_kernel(x_ref, w0_ref, b0_ref, w1_ref, b1_ref, w2_ref, b2_ref,
                ev0_ref, fcw_ref, u_ref, sp_ref, acc_ref):
    k = pl.program_id(0)
    nk = pl.num_programs(0)
    xb = x_ref[...]
    col = jax.lax.broadcasted_iota(jnp.int32, xb.shape, 1) + k * _KT
    xb = jnp.where(col < _N, xb, 0.0)

    sp_ref[...] = jax.lax.dot_general(fcw_ref[:, :_DIM], ev0_ref[...], _DNT,
                                      preferred_element_type=jnp.float32)

    w0b = w0_ref[...]
    wcol = jax.lax.broadcasted_iota(jnp.int32, w0b.shape, 1) + k * _KT
    w0b = jnp.where(wcol < _N, w0b, 0.0)
    part = jax.lax.dot_general(xb, w0b, _DNT,
                               preferred_element_type=jnp.float32)

    @pl.when(k == 0)
    def _():
        acc_ref[...] = part

    @pl.when(k > 0)
    def _():
        acc_ref[...] += part

    @pl.when(k == nk - 1)
    def _():
        h = _selu(acc_ref[...] + b0_ref[...])
        h2 = _selu(jax.lax.dot_general(h, w1_ref[...], _DNT,
                                       preferred_element_type=jnp.float32)
                   + b1_ref[...])
        u_ref[...] = (jax.lax.dot_general(h2, w2_ref[...], _DNT,
                                          preferred_element_type=jnp.float32)
                      + b2_ref[...])


_DNN = (((1,), (0,)), ((), ()))


def _agg_kernel(u_ref, sp_ref, nb0_ref, nb1_ref, nb2_ref, nb3_ref,
                rel0_ref, rel1_ref, rel2_ref, rel3_ref, fcw_ref, fcb_ref,
                out_ref):
    u = u_ref[...]
    us = u * (1.0 / _DIM)
    fb = fcw_ref[:, _DIM:]
    fcb = fcb_ref[...]
    nbk = (nb0_ref, nb1_ref, nb2_ref, nb3_ref)
    relk = (rel0_ref, rel1_ref, rel2_ref, rel3_ref)

    for c in range(_T // _C):
        sl = slice(c * _C, (c + 1) * _C)

        s = [jax.lax.dot_general(us, relk[kk][:, sl], _DNN,
                                 preferred_element_type=jnp.float32)
             for kk in range(_K)]
        m = jnp.maximum(jnp.maximum(s[0], s[1]), jnp.maximum(s[2], s[3]))
        e = [jnp.exp(sk - m) for sk in s]
        rz = 1.0 / (e[0] + e[1] + e[2] + e[3])
        p = [ek * rz for ek in e]

        npk = [jax.lax.dot_general(fb, nbk[kk][:, sl], _DNN,
                                   preferred_element_type=jnp.float32)
               for kk in range(_K)]
        sp = sp_ref[:, sl] + fcb

        acc = jnp.zeros_like(s[0])
        for j in range(_DIM):
            zj = (sp[j:j + 1, :]
                  + p[0] * npk[0][j:j + 1, :] + p[1] * npk[1][j:j + 1, :]
                  + p[2] * npk[2][j:j + 1, :] + p[3] * npk[3][j:j + 1, :])
            acc = acc + u[:, j:j + 1] * jnp.tanh(zj)
        out_ref[:, sl] = 1.0 / (1.0 + jnp.exp(-acc))


@functools.partial(jax.jit, static_argnames=())
def kernel(x, entity_vec0, entity_vec1, relation_vec0, encode_w0, encode_b0,
           encode_w1, encode_b1, enc2u_w, enc2u_b, fc_w, fc_b):
    nk = pl.cdiv(_N, _KT)
    u, sp = pl.pallas_call(
        _enc_kernel,
        grid=(nk,),
        in_specs=[
            pl.BlockSpec((_B, _KT), lambda k: (0, k)),
            pl.BlockSpec((512, _KT), lambda k: (0, k)),
            pl.BlockSpec((1, 512), lambda k: (0, 0)),
            pl.BlockSpec((128, 512), lambda k: (0, 0)),
            pl.BlockSpec((1, 128), lambda k: (0, 0)),
            pl.BlockSpec((_DIM, 128), lambda k: (0, 0)),
            pl.BlockSpec((1, _DIM), lambda k: (0, 0)),
            pl.BlockSpec((_KT, _DIM), lambda k: (k, 0)),
            pl.BlockSpec((_DIM, 2 * _DIM), lambda k: (0, 0)),
        ],
        out_specs=[
            pl.BlockSpec((_B, _DIM), lambda k: (0, 0)),
            pl.BlockSpec((_DIM, _KT), lambda k: (0, k)),
        ],
        out_shape=[
            jax.ShapeDtypeStruct((_B, _DIM), jnp.float32),
            jax.ShapeDtypeStruct((_DIM, _N), jnp.float32),
        ],
        scratch_shapes=[pltpu.VMEM((_B, 512), jnp.float32)],
        compiler_params=pltpu.CompilerParams(
            dimension_semantics=("arbitrary",)),
    )(x, encode_w0, encode_b0.reshape(1, 512), encode_w1,
      encode_b1.reshape(1, 128), enc2u_w, enc2u_b.reshape(1, _DIM),
      entity_vec0, fc_w)

    nt = pl.cdiv(_N, _T)
    npad = nt * _T

    def _sectioned(tab):
        t3 = jnp.pad(tab.reshape(_N, _K, _DIM),
                     ((0, npad - _N), (0, 0), (0, 0)))
        return jnp.transpose(t3, (2, 1, 0)).reshape(_DIM, _K * npad)

    nbt = _sectioned(entity_vec1)
    relt = _sectioned(relation_vec0)

    tab_specs = [pl.BlockSpec((_DIM, _T), lambda i, kk=kk: (0, kk * nt + i))
                 for kk in range(_K)]
    final = pl.pallas_call(
        _agg_kernel,
        grid=(nt,),
        in_specs=[
            pl.BlockSpec((_B, _DIM), lambda i: (0, 0)),
            pl.BlockSpec((_DIM, _T), lambda i: (0, i)),
            *tab_specs,
            *tab_specs,
            pl.BlockSpec((_DIM, 2 * _DIM), lambda i: (0, 0)),
            pl.BlockSpec((_DIM, 1), lambda i: (0, 0)),
        ],
        out_specs=pl.BlockSpec((_B, _T), lambda i: (0, i)),
        out_shape=jax.ShapeDtypeStruct((_B, _N), jnp.float32),
        compiler_params=pltpu.CompilerParams(
            dimension_semantics=("parallel",)),
    )(u, sp, nbt, nbt, nbt, nbt, relt, relt, relt, relt, fc_w,
      fc_b.reshape(_DIM, 1))
    return final


# speedup vs baseline: 1.3170x; 1.3170x over previous
import functools

import jax
import jax.numpy as jnp
from jax.experimental import pallas as pl
from jax.experimental.pallas import tpu as pltpu

_N = 25274
_DIM = 16
_K = 4
_B = 64

_KT = 2048
_T = 1024
_C = 256

_SELU_ALPHA = 1.6732632423543772
_SELU_SCALE = 1.0507009873554805


def _selu(v):
    return _SELU_SCALE * jnp.where(v > 0, v, _SELU_ALPHA * (jnp.exp(v) - 1.0))


_DNT = (((1,), (1,)), ((), ()))


def _enc_kernel(x_ref, w0_ref, b0_ref, w1_ref, b1_ref, w2_ref, b2_ref,
                ev0_ref, fcw_ref, u_ref, sp_ref, acc_ref):
    k = pl.program_id(0)
    nk = pl.num_programs(0)
    xb = x_ref[...]
    col = jax.lax.broadcasted_iota(jnp.int32, xb.shape, 1) + k * _KT
    xb = jnp.where(col < _N, xb, 0.0)

    sp_ref[...] = jax.lax.dot_general(fcw_ref[:, :_DIM], ev0_ref[...], _DNT,
                                      preferred_element_type=jnp.float32)

    w0b = w0_ref[...]
    wcol = jax.lax.broadcasted_iota(jnp.int32, w0b.shape, 1) + k * _KT
    w0b = jnp.where(wcol < _N, w0b, 0.0)
    part = jax.lax.dot_general(xb, w0b, _DNT,
                               preferred_element_type=jnp.float32)

    @pl.when(k == 0)
    def _():
        acc_ref[...] = part

    @pl.when(k > 0)
    def _():
        acc_ref[...] += part

    @pl.when(k == nk - 1)
    def _():
        h = _selu(acc_ref[...] + b0_ref[...])
        h2 = _selu(jax.lax.dot_general(h, w1_ref[...], _DNT,
                                       preferred_element_type=jnp.float32)
                   + b1_ref[...])
        u_ref[...] = (jax.lax.dot_general(h2, w2_ref[...], _DNT,
                                          preferred_element_type=jnp.float32)
                      + b2_ref[...])


def _agg_kernel(u_ref, sp_ref, nb_ref, rel_ref, fcw_ref, fcb_ref, out_ref):
    u = u_ref[...]
    us = u * (1.0 / _DIM)
    fb = fcw_ref[:, _DIM:]
    fcb = fcb_ref[...]

    for c in range(_T // _C):
        sl = slice(c * _C, (c + 1) * _C)
        rk = [rel_ref[pl.Slice(c * _C * _K + kk, _C, _K), :]
              for kk in range(_K)]
        nk_ = [nb_ref[pl.Slice(c * _C * _K + kk, _C, _K), :]
               for kk in range(_K)]

        s = [jax.lax.dot_general(us, rk[kk], _DNT,
                                 preferred_element_type=jnp.float32)
             for kk in range(_K)]
        m = jnp.maximum(jnp.maximum(s[0], s[1]), jnp.maximum(s[2], s[3]))
        e = [jnp.exp(sk - m) for sk in s]
        rz = 1.0 / (e[0] + e[1] + e[2] + e[3])
        p = [ek * rz for ek in e]

        npk = [jax.lax.dot_general(fb, nk_[kk], _DNT,
                                   preferred_element_type=jnp.float32)
               for kk in range(_K)]
        sp = sp_ref[:, sl] + fcb

        acc = jnp.zeros_like(s[0])
        for j in range(_DIM):
            zj = (sp[j:j + 1, :]
                  + p[0] * npk[0][j:j + 1, :] + p[1] * npk[1][j:j + 1, :]
                  + p[2] * npk[2][j:j + 1, :] + p[3] * npk[3][j:j + 1, :])
            acc = acc + u[:, j:j + 1] * jnp.tanh(zj)
        out_ref[:, sl] = 1.0 / (1.0 + jnp.exp(-acc))


@functools.partial(jax.jit, static_argnames=())
def kernel(x, entity_vec0, entity_vec1, relation_vec0, encode_w0, encode_b0,
           encode_w1, encode_b1, enc2u_w, enc2u_b, fc_w, fc_b):
    nk = pl.cdiv(_N, _KT)
    u, sp = pl.pallas_call(
        _enc_kernel,
        grid=(nk,),
        in_specs=[
            pl.BlockSpec((_B, _KT), lambda k: (0, k)),
            pl.BlockSpec((512, _KT), lambda k: (0, k)),
            pl.BlockSpec((1, 512), lambda k: (0, 0)),
            pl.BlockSpec((128, 512), lambda k: (0, 0)),
            pl.BlockSpec((1, 128), lambda k: (0, 0)),
            pl.BlockSpec((_DIM, 128), lambda k: (0, 0)),
            pl.BlockSpec((1, _DIM), lambda k: (0, 0)),
            pl.BlockSpec((_KT, _DIM), lambda k: (k, 0)),
            pl.BlockSpec((_DIM, 2 * _DIM), lambda k: (0, 0)),
        ],
        out_specs=[
            pl.BlockSpec((_B, _DIM), lambda k: (0, 0)),
            pl.BlockSpec((_DIM, _KT), lambda k: (0, k)),
        ],
        out_shape=[
            jax.ShapeDtypeStruct((_B, _DIM), jnp.float32),
            jax.ShapeDtypeStruct((_DIM, _N), jnp.float32),
        ],
        scratch_shapes=[pltpu.VMEM((_B, 512), jnp.float32)],
        compiler_params=pltpu.CompilerParams(
            dimension_semantics=("arbitrary",)),
    )(x, encode_w0, encode_b0.reshape(1, 512), encode_w1,
      encode_b1.reshape(1, 128), enc2u_w, enc2u_b.reshape(1, _DIM),
      entity_vec0, fc_w)

    nt = pl.cdiv(_N, _T)
    final = pl.pallas_call(
        _agg_kernel,
        grid=(nt,),
        in_specs=[
            pl.BlockSpec((_B, _DIM), lambda i: (0, 0)),
            pl.BlockSpec((_DIM, _T), lambda i: (0, i)),
            pl.BlockSpec((_T * _K, _DIM), lambda i: (i, 0)),
            pl.BlockSpec((_T * _K, _DIM), lambda i: (i, 0)),
            pl.BlockSpec((_DIM, 2 * _DIM), lambda i: (0, 0)),
            pl.BlockSpec((_DIM, 1), lambda i: (0, 0)),
        ],
        out_specs=pl.BlockSpec((_B, _T), lambda i: (0, i)),
        out_shape=jax.ShapeDtypeStruct((_B, _N), jnp.float32),
        compiler_params=pltpu.CompilerParams(
            dimension_semantics=("parallel",)),
    )(u, sp, entity_vec1, relation_vec0, fc_w, fc_b.reshape(_DIM, 1))
    return final


# speedup vs baseline: 1.3194x; 1.0018x over previous
import functools

import jax
import jax.numpy as jnp
from jax.experimental import pallas as pl
from jax.experimental.pallas import tpu as pltpu

_N = 25274
_DIM = 16
_K = 4
_B = 64

_KT = 4096
_T = 2048
_C = 256

_SELU_ALPHA = 1.6732632423543772
_SELU_SCALE = 1.0507009873554805


def _selu(v):
    return _SELU_SCALE * jnp.where(v > 0, v, _SELU_ALPHA * (jnp.exp(v) - 1.0))


_DNT = (((1,), (1,)), ((), ()))


def _enc_kernel(x_ref, w0_ref, b0_ref, w1_ref, b1_ref, w2_ref, b2_ref,
                ev0_ref, fcw_ref, u_ref, sp_ref, acc_ref):
    k = pl.program_id(0)
    nk = pl.num_programs(0)
    xb = x_ref[...]
    col = jax.lax.broadcasted_iota(jnp.int32, xb.shape, 1) + k * _KT
    xb = jnp.where(col < _N, xb, 0.0)

    sp_ref[...] = jax.lax.dot_general(fcw_ref[:, :_DIM], ev0_ref[...], _DNT,
                                      preferred_element_type=jnp.float32)

    w0b = w0_ref[...]
    wcol = jax.lax.broadcasted_iota(jnp.int32, w0b.shape, 1) + k * _KT
    w0b = jnp.where(wcol < _N, w0b, 0.0)
    part = jax.lax.dot_general(xb, w0b, _DNT,
                               preferred_element_type=jnp.float32)

    @pl.when(k == 0)
    def _():
        acc_ref[...] = part

    @pl.when(k > 0)
    def _():
        acc_ref[...] += part

    @pl.when(k == nk - 1)
    def _():
        h = _selu(acc_ref[...] + b0_ref[...])
        h2 = _selu(jax.lax.dot_general(h, w1_ref[...], _DNT,
                                       preferred_element_type=jnp.float32)
                   + b1_ref[...])
        u_ref[...] = (jax.lax.dot_general(h2, w2_ref[...], _DNT,
                                          preferred_element_type=jnp.float32)
                      + b2_ref[...])


def _agg_kernel(u_ref, sp_ref, nb_ref, rel_ref, fcw_ref, fcb_ref, out_ref):
    u = u_ref[...]
    us = u * (1.0 / _DIM)
    fb = fcw_ref[:, _DIM:]
    fcb = fcb_ref[...]

    for c in range(_T // _C):
        sl = slice(c * _C, (c + 1) * _C)
        rk = [rel_ref[pl.Slice(c * _C * _K + kk, _C, _K), :]
              for kk in range(_K)]
        nk_ = [nb_ref[pl.Slice(c * _C * _K + kk, _C, _K), :]
               for kk in range(_K)]

        s = [jax.lax.dot_general(us, rk[kk], _DNT,
                                 preferred_element_type=jnp.float32)
             for kk in range(_K)]
        m = jnp.maximum(jnp.maximum(s[0], s[1]), jnp.maximum(s[2], s[3]))
        e = [jnp.exp(sk - m) for sk in s]
        rz = 1.0 / (e[0] + e[1] + e[2] + e[3])
        p = [ek * rz for ek in e]

        npk = [jax.lax.dot_general(fb, nk_[kk], _DNT,
                                   preferred_element_type=jnp.float32)
               for kk in range(_K)]
        sp = sp_ref[:, sl] + fcb

        acc = jnp.zeros_like(s[0])
        for j in range(_DIM):
            zj = (sp[j:j + 1, :]
                  + p[0] * npk[0][j:j + 1, :] + p[1] * npk[1][j:j + 1, :]
                  + p[2] * npk[2][j:j + 1, :] + p[3] * npk[3][j:j + 1, :])
            acc = acc + u[:, j:j + 1] * jnp.tanh(zj)
        out_ref[:, sl] = 1.0 / (1.0 + jnp.exp(-acc))


@functools.partial(jax.jit, static_argnames=())
def kernel(x, entity_vec0, entity_vec1, relation_vec0, encode_w0, encode_b0,
           encode_w1, encode_b1, enc2u_w, enc2u_b, fc_w, fc_b):
    nk = pl.cdiv(_N, _KT)
    u, sp = pl.pallas_call(
        _enc_kernel,
        grid=(nk,),
        in_specs=[
            pl.BlockSpec((_B, _KT), lambda k: (0, k)),
            pl.BlockSpec((512, _KT), lambda k: (0, k)),
            pl.BlockSpec((1, 512), lambda k: (0, 0)),
            pl.BlockSpec((128, 512), lambda k: (0, 0)),
            pl.BlockSpec((1, 128), lambda k: (0, 0)),
            pl.BlockSpec((_DIM, 128), lambda k: (0, 0)),
            pl.BlockSpec((1, _DIM), lambda k: (0, 0)),
            pl.BlockSpec((_KT, _DIM), lambda k: (k, 0)),
            pl.BlockSpec((_DIM, 2 * _DIM), lambda k: (0, 0)),
        ],
        out_specs=[
            pl.BlockSpec((_B, _DIM), lambda k: (0, 0)),
            pl.BlockSpec((_DIM, _KT), lambda k: (0, k)),
        ],
        out_shape=[
            jax.ShapeDtypeStruct((_B, _DIM), jnp.float32),
            jax.ShapeDtypeStruct((_DIM, _N), jnp.float32),
        ],
        scratch_shapes=[pltpu.VMEM((_B, 512), jnp.float32)],
        compiler_params=pltpu.CompilerParams(
            dimension_semantics=("arbitrary",)),
    )(x, encode_w0, encode_b0.reshape(1, 512), encode_w1,
      encode_b1.reshape(1, 128), enc2u_w, enc2u_b.reshape(1, _DIM),
      entity_vec0, fc_w)

    nt = pl.cdiv(_N, _T)
    final = pl.pallas_call(
        _agg_kernel,
        grid=(nt,),
        in_specs=[
            pl.BlockSpec((_B, _DIM), lambda i: (0, 0)),
            pl.BlockSpec((_DIM, _T), lambda i: (0, i)),
            pl.BlockSpec((_T * _K, _DIM), lambda i: (i, 0)),
            pl.BlockSpec((_T * _K, _DIM), lambda i: (i, 0)),
            pl.BlockSpec((_DIM, 2 * _DIM), lambda i: (0, 0)),
            pl.BlockSpec((_DIM, 1), lambda i: (0, 0)),
        ],
        out_specs=pl.BlockSpec((_B, _T), lambda i: (0, i)),
        out_shape=jax.ShapeDtypeStruct((_B, _N), jnp.float32),
        compiler_params=pltpu.CompilerParams(
            dimension_semantics=("parallel",)),
    )(u, sp, entity_vec1, relation_vec0, fc_w, fc_b.reshape(_DIM, 1))
    return final
